# baseline (speedup 1.0000x reference)
import functools

import jax
import jax.numpy as jnp
from jax import lax
from jax.experimental import pallas as pl
from jax.experimental.pallas import tpu as pltpu
from jax.experimental.pallas import tpu_sc as plsc

B = 16384
F = 26
V = 100000
D = 16

NC = 2
NS = 16
NW = NC * NS

NROW = F * V
NBLK_FULL = NROW // 128
TAIL = NROW - NBLK_FULL * 128
TAIL_WID = NBLK_FULL % NW

ROWS_PER_W = B // NW
CHUNK_ROWS = 64
N_CHUNKS = ROWS_PER_W // CHUNK_ROWS
CHUNK_IDX = CHUNK_ROWS * F
GATHER_N = 128
N_GATHERS = CHUNK_IDX // GATHER_N
W_PAD = 32


def _relayout_body(emb_hbm, tail_hbm, out_hbm, i0, i1, o0, o1,
                   si0, si1, so0, so1):
  wid = lax.axis_index("s") * NC + lax.axis_index("c")
  ibufs = (i0, i1)
  obufs = (o0, o1)
  isems = (si0, si1)
  osems = (so0, so1)
  lane = lax.iota(jnp.int32, 16)

  nblk_w = (NBLK_FULL - wid + NW - 1) // NW

  @pl.when(wid == TAIL_WID)
  def _():
    pltpu.sync_copy(tail_hbm,
                    out_hbm.at[pl.ds(NBLK_FULL * 128 * D, TAIL * D)])

  def fire_in(slot, k):
    @pl.when(k < nblk_w)
    def _():
      cb = wid + k * NW
      pltpu.async_copy(emb_hbm.at[:, pl.ds(cb * 128, 128)],
                       ibufs[slot].at[:, pl.ds(0, 128)], isems[slot])

  def step(slot, k):
    @pl.when(k < nblk_w)
    def _():
      cb = wid + k * NW
      pltpu.make_async_copy(emb_hbm.at[:, pl.ds(cb * 128, 128)],
                            ibufs[slot].at[:, pl.ds(0, 128)],
                            isems[slot]).wait()

      @pl.when(k >= 2)
      def _():
        pltpu.make_async_copy(obufs[slot],
                              out_hbm.at[pl.ds(cb * 2048, 2048)],
                              osems[slot]).wait()

      for j in range(128):
        col = plsc.load_gather(ibufs[slot],
                               [lane, jnp.full((16,), j, jnp.int32)])
        obufs[slot][pl.ds(j * 16, 16)] = col
      pltpu.async_copy(obufs[slot], out_hbm.at[pl.ds(cb * 2048, 2048)],
                       osems[slot])
      fire_in(slot, k + 2)

  fire_in(0, 0)
  fire_in(1, 1)

  def pair(h, _):
    step(0, 2 * h)
    step(1, 2 * h + 1)
    return 0

  lax.fori_loop(0, (NBLK_FULL // NW + 2) // 2, pair, 0)

  pltpu.make_async_copy(o0, out_hbm.at[pl.ds(0, 2048)], so0).wait()
  pltpu.make_async_copy(o1, out_hbm.at[pl.ds(0, 2048)], so1).wait()


@functools.partial(jax.jit, static_argnames=())
def _relayout_call(emb_t, tail):
  mesh = plsc.VectorSubcoreMesh(core_axis_name="c", subcore_axis_name="s")
  run = pl.kernel(
      _relayout_body,
      out_type=jax.ShapeDtypeStruct((NROW * D,), jnp.float32),
      mesh=mesh,
      compiler_params=pltpu.CompilerParams(
          needs_layout_passes=False, use_tc_tiling_on_sc=True),
      scratch_types=[
          pltpu.VMEM((16, 129), jnp.float32),
          pltpu.VMEM((16, 129), jnp.float32),
          pltpu.VMEM((2048,), jnp.float32),
          pltpu.VMEM((2048,), jnp.float32),
          pltpu.SemaphoreType.DMA,
          pltpu.SemaphoreType.DMA,
          pltpu.SemaphoreType.DMA,
          pltpu.SemaphoreType.DMA,
      ],
  )
  return run(emb_t, tail)


def _fm_body(idx_hbm, emb_hbm, w_hbm, bias_hbm, out_hbm,
             idx0, idx1, e0, e1, w0, w1, out_v, bias_v, tbuf,
             se0, se1, sw0, sw1):
  wid = lax.axis_index("s") * NC + lax.axis_index("c")
  base_idx = wid * (ROWS_PER_W * F)
  idx_bufs = (idx0, idx1)
  e_bufs = (e0, e1)
  w_bufs = (w0, w1)
  e_sems = (se0, se1)
  w_sems = (sw0, sw1)

  lane = lax.iota(jnp.int32, 16)
  wmask = lane < (F - 16)

  pltpu.sync_copy(bias_hbm, bias_v)

  def fire(slot, c):
    idx_v = idx_bufs[slot]
    off = base_idx + c * CHUNK_IDX
    pltpu.sync_copy(idx_hbm.at[pl.ds(off, CHUNK_IDX)], idx_v)

    def add_off(k, _):
      p = k * 16
      v = idx_v[pl.ds(p, 16)]
      fld = lax.rem(p + lane, F)
      idx_v[pl.ds(p, 16)] = v + fld * V
      return 0

    lax.fori_loop(0, CHUNK_IDX // 16, add_off, 0)

    def fire_one(j, _):
      isl = idx_v.at[pl.ds(j * GATHER_N, GATHER_N)]
      pltpu.async_copy(emb_hbm.at[isl],
                       e_bufs[slot].at[pl.ds(j * GATHER_N, GATHER_N)],
                       e_sems[slot])
      pltpu.async_copy(w_hbm.at[isl],
                       w_bufs[slot].at[pl.ds(j * GATHER_N, GATHER_N)],
                       w_sems[slot])
      return 0

    lax.fori_loop(0, N_GATHERS, fire_one, 0)

  def drain(slot):
    idx_v = idx_bufs[slot]

    def dj(j, _):
      isl = idx_v.at[pl.ds(j * GATHER_N, GATHER_N)]
      pltpu.make_async_copy(emb_hbm.at[isl],
                            e_bufs[slot].at[pl.ds(j * GATHER_N, GATHER_N)],
                            e_sems[slot]).wait()
      pltpu.make_async_copy(w_hbm.at[isl],
                            w_bufs[slot].at[pl.ds(j * GATHER_N, GATHER_N)],
                            w_sems[slot]).wait()
      return 0

    lax.fori_loop(0, N_GATHERS, dj, 0)

  def compute(slot, c):
    e_v = e_bufs[slot]
    w_v = w_bufs[slot]

    def group(g, _):
      def row(r, _):
        b = g * 16 + r
        rbase = b * F
        acc_s = jnp.zeros((16,), jnp.float32)
        acc_q = jnp.zeros((16,), jnp.float32)
        for f in range(F):
          e = e_v[rbase + f, :]
          acc_s = acc_s + e
          acc_q = acc_q + e * e
        d = acc_s * acc_s - acc_q
        wv1 = w_v[pl.ds(rbase, 16)]
        wv2 = w_v[pl.ds(rbase + 16, 16)]
        t = 0.5 * d + wv1 + jnp.where(wmask, wv2, 0.0)
        tbuf[r, pl.ds(0, 16)] = t
        return 0

      lax.fori_loop(0, 16, row, 0)
      acc = jnp.zeros((16,), jnp.float32)
      for dcol in range(16):
        col = plsc.load_gather(tbuf, [lane, jnp.full((16,), dcol, jnp.int32)])
        acc = acc + col
      out_v[pl.ds(c * CHUNK_ROWS + g * 16, 16)] = acc
      return 0

    lax.fori_loop(0, CHUNK_ROWS // 16, group, 0)

  fire(0, 0)
  for c in range(N_CHUNKS):
    slot = c % 2
    if c + 1 < N_CHUNKS:
      fire(1 - slot, c + 1)
    drain(slot)
    compute(slot, c)

  bias = bias_v[...]

  def sig(i, _):
    v = out_v[pl.ds(i * 16, 16)]
    z = v + bias
    out_v[pl.ds(i * 16, 16)] = 1.0 / (1.0 + jnp.exp(-z))
    return 0

  lax.fori_loop(0, ROWS_PER_W // 16, sig, 0)
  pltpu.sync_copy(out_v, out_hbm.at[pl.ds(wid * ROWS_PER_W, ROWS_PER_W)])


@functools.partial(jax.jit, static_argnames=())
def _fm_call(idx, emb_table, w_flat, bias16):
  mesh = plsc.VectorSubcoreMesh(core_axis_name="c", subcore_axis_name="s")
  run = pl.kernel(
      _fm_body,
      out_type=jax.ShapeDtypeStruct((B,), jnp.float32),
      mesh=mesh,
      compiler_params=pltpu.CompilerParams(
          needs_layout_passes=False, use_tc_tiling_on_sc=False),
      scratch_types=[
          pltpu.VMEM((CHUNK_IDX,), jnp.int32),
          pltpu.VMEM((CHUNK_IDX,), jnp.int32),
          pltpu.VMEM((CHUNK_IDX, D), jnp.float32),
          pltpu.VMEM((CHUNK_IDX, D), jnp.float32),
          pltpu.VMEM((CHUNK_IDX + W_PAD,), jnp.float32),
          pltpu.VMEM((CHUNK_IDX + W_PAD,), jnp.float32),
          pltpu.VMEM((ROWS_PER_W,), jnp.float32),
          pltpu.VMEM((16,), jnp.float32),
          pltpu.VMEM((16, 17), jnp.float32),
          pltpu.SemaphoreType.DMA,
          pltpu.SemaphoreType.DMA,
          pltpu.SemaphoreType.DMA,
          pltpu.SemaphoreType.DMA,
      ],
  )
  return run(idx, emb_table, w_flat, bias16)


def kernel(x, emb_table, w_table, bias):
  idx = x.reshape(-1)
  tail = emb_table[NBLK_FULL * 128:].reshape(-1)
  flat = _relayout_call(emb_table.T, tail)
  table = flat.reshape(NROW, D)
  w_flat = w_table.reshape(-1)
  bias16 = jnp.broadcast_to(bias, (16,))
  out = _fm_call(idx, table, w_flat, bias16)
  return out.reshape(B, 1)


# speedup vs baseline: 2.3228x; 2.3228x over previous
import functools

import jax
import jax.numpy as jnp
from jax import lax
from jax.experimental import pallas as pl
from jax.experimental.pallas import tpu as pltpu
from jax.experimental.pallas import tpu_sc as plsc

B = 16384
F = 26
V = 100000
D = 16

NC = 2
NS = 16
NW = NC * NS

NROW = F * V
NBLK_FULL = NROW // 128
TAIL = NROW - NBLK_FULL * 128
TAIL_WID = NBLK_FULL % NW

ROWS_PER_W = B // NW
CHUNK_ROWS = 64
N_CHUNKS = ROWS_PER_W // CHUNK_ROWS
CHUNK_IDX = CHUNK_ROWS * F
GATHER_N = 128
N_GATHERS = CHUNK_IDX // GATHER_N
W_PAD = 32


def _relayout_body(emb_hbm, tail_hbm, out_hbm, i0, i1, o0, o1,
                   si0, si1, so0, so1):
  wid = lax.axis_index("s") * NC + lax.axis_index("c")
  ibufs = (i0, i1)
  obufs = (o0, o1)
  isems = (si0, si1)
  osems = (so0, so1)
  lane = lax.iota(jnp.int32, 16)
  lane16 = lane * 16

  nblk_w = (NBLK_FULL - wid + NW - 1) // NW

  @pl.when(wid == TAIL_WID)
  def _():
    pltpu.sync_copy(tail_hbm,
                    out_hbm.at[pl.ds(NBLK_FULL * 128 * D, TAIL * D)])

  def fire_in(slot, k):
    @pl.when(k < nblk_w)
    def _():
      cb = wid + k * NW
      pltpu.async_copy(emb_hbm.at[:, pl.ds(cb * 128, 128)], ibufs[slot],
                       isems[slot])

  def step(slot, k):
    @pl.when(k < nblk_w)
    def _():
      cb = wid + k * NW
      pltpu.make_async_copy(emb_hbm.at[:, pl.ds(cb * 128, 128)], ibufs[slot],
                            isems[slot]).wait()

      @pl.when(k >= 2)
      def _():
        pltpu.make_async_copy(obufs[slot],
                              out_hbm.at[pl.ds(cb * 2048, 2048)],
                              osems[slot]).wait()

      for d in range(16):
        for j0 in range(0, 128, 16):
          v = ibufs[slot][d, pl.ds(j0, 16)]
          plsc.store_scatter(obufs[slot], [lane16 + (j0 * 16 + d)], v)
      pltpu.async_copy(obufs[slot], out_hbm.at[pl.ds(cb * 2048, 2048)],
                       osems[slot])
      fire_in(slot, k + 2)

  fire_in(0, 0)
  fire_in(1, 1)

  def pair(h, _):
    step(0, 2 * h)
    step(1, 2 * h + 1)
    return 0

  lax.fori_loop(0, (NBLK_FULL // NW + 2) // 2, pair, 0)

  pltpu.make_async_copy(o0, out_hbm.at[pl.ds(0, 2048)], so0).wait()
  pltpu.make_async_copy(o1, out_hbm.at[pl.ds(0, 2048)], so1).wait()


@functools.partial(jax.jit, static_argnames=())
def _relayout_call(emb_t, tail):
  mesh = plsc.VectorSubcoreMesh(core_axis_name="c", subcore_axis_name="s")
  run = pl.kernel(
      _relayout_body,
      out_type=jax.ShapeDtypeStruct((NROW * D,), jnp.float32),
      mesh=mesh,
      compiler_params=pltpu.CompilerParams(
          needs_layout_passes=False, use_tc_tiling_on_sc=True),
      scratch_types=[
          pltpu.VMEM((16, 128), jnp.float32),
          pltpu.VMEM((16, 128), jnp.float32),
          pltpu.VMEM((2048,), jnp.float32),
          pltpu.VMEM((2048,), jnp.float32),
          pltpu.SemaphoreType.DMA,
          pltpu.SemaphoreType.DMA,
          pltpu.SemaphoreType.DMA,
          pltpu.SemaphoreType.DMA,
      ],
  )
  return run(emb_t, tail)


def _fm_body(idx_hbm, emb_hbm, w_hbm, bias_hbm, out_hbm,
             idx0, idx1, e0, e1, w0, w1, out_v, bias_v, tbuf,
             se0, se1, sw0, sw1):
  wid = lax.axis_index("s") * NC + lax.axis_index("c")
  base_idx = wid * (ROWS_PER_W * F)
  idx_bufs = (idx0, idx1)
  e_bufs = (e0, e1)
  w_bufs = (w0, w1)
  e_sems = (se0, se1)
  w_sems = (sw0, sw1)

  lane = lax.iota(jnp.int32, 16)
  wmask = lane < (F - 16)

  pltpu.sync_copy(bias_hbm, bias_v)

  def fire(slot, c):
    idx_v = idx_bufs[slot]
    off = base_idx + c * CHUNK_IDX
    pltpu.sync_copy(idx_hbm.at[pl.ds(off, CHUNK_IDX)], idx_v)

    def add_off(k, _):
      p = k * 16
      v = idx_v[pl.ds(p, 16)]
      fld = lax.rem(p + lane, F)
      idx_v[pl.ds(p, 16)] = v + fld * V
      return 0

    lax.fori_loop(0, CHUNK_IDX // 16, add_off, 0)

    def fire_one(j, _):
      isl = idx_v.at[pl.ds(j * GATHER_N, GATHER_N)]
      pltpu.async_copy(emb_hbm.at[isl],
                       e_bufs[slot].at[pl.ds(j * GATHER_N, GATHER_N)],
                       e_sems[slot])
      pltpu.async_copy(w_hbm.at[isl],
                       w_bufs[slot].at[pl.ds(j * GATHER_N, GATHER_N)],
                       w_sems[slot])
      return 0

    lax.fori_loop(0, N_GATHERS, fire_one, 0)

  def drain(slot):
    idx_v = idx_bufs[slot]

    def dj(j, _):
      isl = idx_v.at[pl.ds(j * GATHER_N, GATHER_N)]
      pltpu.make_async_copy(emb_hbm.at[isl],
                            e_bufs[slot].at[pl.ds(j * GATHER_N, GATHER_N)],
                            e_sems[slot]).wait()
      pltpu.make_async_copy(w_hbm.at[isl],
                            w_bufs[slot].at[pl.ds(j * GATHER_N, GATHER_N)],
                            w_sems[slot]).wait()
      return 0

    lax.fori_loop(0, N_GATHERS, dj, 0)

  def compute(slot, c):
    e_v = e_bufs[slot]
    w_v = w_bufs[slot]

    def group(g, _):
      def row(r, _):
        b = g * 16 + r
        rbase = b * F
        acc_s = jnp.zeros((16,), jnp.float32)
        acc_q = jnp.zeros((16,), jnp.float32)
        for f in range(F):
          e = e_v[rbase + f, :]
          acc_s = acc_s + e
          acc_q = acc_q + e * e
        d = acc_s * acc_s - acc_q
        wv1 = w_v[pl.ds(rbase, 16)]
        wv2 = w_v[pl.ds(rbase + 16, 16)]
        t = 0.5 * d + wv1 + jnp.where(wmask, wv2, 0.0)
        tbuf[r, pl.ds(0, 16)] = t
        return 0

      lax.fori_loop(0, 16, row, 0)
      acc = jnp.zeros((16,), jnp.float32)
      for dcol in range(16):
        col = plsc.load_gather(tbuf, [lane, jnp.full((16,), dcol, jnp.int32)])
        acc = acc + col
      out_v[pl.ds(c * CHUNK_ROWS + g * 16, 16)] = acc
      return 0

    lax.fori_loop(0, CHUNK_ROWS // 16, group, 0)

  fire(0, 0)
  for c in range(N_CHUNKS):
    slot = c % 2
    if c + 1 < N_CHUNKS:
      fire(1 - slot, c + 1)
    drain(slot)
    compute(slot, c)

  bias = bias_v[...]

  def sig(i, _):
    v = out_v[pl.ds(i * 16, 16)]
    z = v + bias
    out_v[pl.ds(i * 16, 16)] = 1.0 / (1.0 + jnp.exp(-z))
    return 0

  lax.fori_loop(0, ROWS_PER_W // 16, sig, 0)
  pltpu.sync_copy(out_v, out_hbm.at[pl.ds(wid * ROWS_PER_W, ROWS_PER_W)])


@functools.partial(jax.jit, static_argnames=())
def _fm_call(idx, emb_table, w_flat, bias16):
  mesh = plsc.VectorSubcoreMesh(core_axis_name="c", subcore_axis_name="s")
  run = pl.kernel(
      _fm_body,
      out_type=jax.ShapeDtypeStruct((B,), jnp.float32),
      mesh=mesh,
      compiler_params=pltpu.CompilerParams(
          needs_layout_passes=False, use_tc_tiling_on_sc=False),
      scratch_types=[
          pltpu.VMEM((CHUNK_IDX,), jnp.int32),
          pltpu.VMEM((CHUNK_IDX,), jnp.int32),
          pltpu.VMEM((CHUNK_IDX, D), jnp.float32),
          pltpu.VMEM((CHUNK_IDX, D), jnp.float32),
          pltpu.VMEM((CHUNK_IDX + W_PAD,), jnp.float32),
          pltpu.VMEM((CHUNK_IDX + W_PAD,), jnp.float32),
          pltpu.VMEM((ROWS_PER_W,), jnp.float32),
          pltpu.VMEM((16,), jnp.float32),
          pltpu.VMEM((16, 17), jnp.float32),
          pltpu.SemaphoreType.DMA,
          pltpu.SemaphoreType.DMA,
          pltpu.SemaphoreType.DMA,
          pltpu.SemaphoreType.DMA,
      ],
  )
  return run(idx, emb_table, w_flat, bias16)


def kernel(x, emb_table, w_table, bias):
  idx = x.reshape(-1)
  tail = emb_table[NBLK_FULL * 128:].reshape(-1)
  flat = _relayout_call(emb_table.T, tail)
  table = flat.reshape(NROW, D)
  w_flat = w_table.reshape(-1)
  bias16 = jnp.broadcast_to(bias, (16,))
  out = _fm_call(idx, table, w_flat, bias16)
  return out.reshape(B, 1)


# speedup vs baseline: 2.4463x; 1.0532x over previous
import functools

import jax
import jax.numpy as jnp
from jax import lax
from jax.experimental import pallas as pl
from jax.experimental.pallas import tpu as pltpu
from jax.experimental.pallas import tpu_sc as plsc

B = 16384
F = 26
V = 100000
D = 16

NC = 2
NS = 16
NW = NC * NS

NROW = F * V
NBLK_FULL = NROW // 128
TAIL = NROW - NBLK_FULL * 128
TAIL_WID = NBLK_FULL % NW
SBC = 512
NSB = NROW // SBC

ROWS_PER_W = B // NW
CHUNK_ROWS = 64
N_CHUNKS = ROWS_PER_W // CHUNK_ROWS
CHUNK_IDX = CHUNK_ROWS * F
GATHER_N = 128
N_GATHERS = CHUNK_IDX // GATHER_N
W_PAD = 32


def _relayout_body(emb_hbm, tail_hbm, out_hbm, i0, i1, o0, o1,
                   si0, si1, so0, so1):
  wid = lax.axis_index("s") * NC + lax.axis_index("c")
  ibufs = (i0, i1)
  obufs = (o0, o1)
  isems = (si0, si1)
  osems = (so0, so1)
  lane = lax.iota(jnp.int32, 16)
  lane16 = lane * 16

  nblk_w = (NSB - wid + NW - 1) // NW

  @pl.when(wid == TAIL_WID)
  def _():
    pltpu.sync_copy(tail_hbm,
                    out_hbm.at[pl.ds(NBLK_FULL * 128 * D, TAIL * D)])

  def fire_in(slot, k):
    @pl.when(k < nblk_w)
    def _():
      cb = wid + k * NW
      pltpu.async_copy(emb_hbm.at[:, pl.ds(cb * SBC, SBC)], ibufs[slot],
                       isems[slot])

  def step(slot, k):
    @pl.when(k < nblk_w)
    def _():
      cb = wid + k * NW
      pltpu.make_async_copy(emb_hbm.at[:, pl.ds(cb * SBC, SBC)], ibufs[slot],
                            isems[slot]).wait()

      @pl.when(k >= 2)
      def _():
        pltpu.make_async_copy(obufs[slot],
                              out_hbm.at[pl.ds(cb * (SBC * 16), SBC * 16)],
                              osems[slot]).wait()

      for d in range(16):
        for j0 in range(0, SBC, 16):
          v = ibufs[slot][d, pl.ds(j0, 16)]
          plsc.store_scatter(obufs[slot], [lane16 + (j0 * 16 + d)], v)
      pltpu.async_copy(obufs[slot], out_hbm.at[pl.ds(cb * (SBC * 16), SBC * 16)],
                       osems[slot])
      fire_in(slot, k + 2)

  fire_in(0, 0)
  fire_in(1, 1)

  def pair(h, _):
    step(0, 2 * h)
    step(1, 2 * h + 1)
    return 0

  lax.fori_loop(0, (NSB // NW + 2) // 2, pair, 0)

  pltpu.make_async_copy(o0, out_hbm.at[pl.ds(0, SBC * 16)], so0).wait()
  pltpu.make_async_copy(o1, out_hbm.at[pl.ds(0, SBC * 16)], so1).wait()


@functools.partial(jax.jit, static_argnames=())
def _relayout_call(emb_t, tail):
  mesh = plsc.VectorSubcoreMesh(core_axis_name="c", subcore_axis_name="s")
  run = pl.kernel(
      _relayout_body,
      out_type=jax.ShapeDtypeStruct((NROW * D,), jnp.float32),
      mesh=mesh,
      compiler_params=pltpu.CompilerParams(
          needs_layout_passes=False, use_tc_tiling_on_sc=True),
      scratch_types=[
          pltpu.VMEM((16, SBC), jnp.float32),
          pltpu.VMEM((16, SBC), jnp.float32),
          pltpu.VMEM((SBC * 16,), jnp.float32),
          pltpu.VMEM((SBC * 16,), jnp.float32),
          pltpu.SemaphoreType.DMA,
          pltpu.SemaphoreType.DMA,
          pltpu.SemaphoreType.DMA,
          pltpu.SemaphoreType.DMA,
      ],
  )
  return run(emb_t, tail)


def _fm_body(idx_hbm, emb_hbm, w_hbm, bias_hbm, out_hbm,
             idx0, idx1, e0, e1, w0, w1, out_v, bias_v, tbuf,
             se0, se1, sw0, sw1):
  wid = lax.axis_index("s") * NC + lax.axis_index("c")
  base_idx = wid * (ROWS_PER_W * F)
  idx_bufs = (idx0, idx1)
  e_bufs = (e0, e1)
  w_bufs = (w0, w1)
  e_sems = (se0, se1)
  w_sems = (sw0, sw1)

  lane = lax.iota(jnp.int32, 16)
  wmask = lane < (F - 16)

  pltpu.sync_copy(bias_hbm, bias_v)

  def fire(slot, c):
    idx_v = idx_bufs[slot]
    off = base_idx + c * CHUNK_IDX
    pltpu.sync_copy(idx_hbm.at[pl.ds(off, CHUNK_IDX)], idx_v)

    def add_off(k, _):
      p = k * 16
      v = idx_v[pl.ds(p, 16)]
      fld = lax.rem(p + lane, F)
      idx_v[pl.ds(p, 16)] = v + fld * V
      return 0

    lax.fori_loop(0, CHUNK_IDX // 16, add_off, 0)

    def fire_one(j, _):
      isl = idx_v.at[pl.ds(j * GATHER_N, GATHER_N)]
      pltpu.async_copy(emb_hbm.at[isl],
                       e_bufs[slot].at[pl.ds(j * GATHER_N, GATHER_N)],
                       e_sems[slot])
      pltpu.async_copy(w_hbm.at[isl],
                       w_bufs[slot].at[pl.ds(j * GATHER_N, GATHER_N)],
                       w_sems[slot])
      return 0

    lax.fori_loop(0, N_GATHERS, fire_one, 0)

  def drain(slot):
    idx_v = idx_bufs[slot]

    def dj(j, _):
      isl = idx_v.at[pl.ds(j * GATHER_N, GATHER_N)]
      pltpu.make_async_copy(emb_hbm.at[isl],
                            e_bufs[slot].at[pl.ds(j * GATHER_N, GATHER_N)],
                            e_sems[slot]).wait()
      pltpu.make_async_copy(w_hbm.at[isl],
                            w_bufs[slot].at[pl.ds(j * GATHER_N, GATHER_N)],
                            w_sems[slot]).wait()
      return 0

    lax.fori_loop(0, N_GATHERS, dj, 0)

  def compute(slot, c):
    e_v = e_bufs[slot]
    w_v = w_bufs[slot]

    def group(g, _):
      def row(r, _):
        b = g * 16 + r
        rbase = b * F
        acc_s = jnp.zeros((16,), jnp.float32)
        acc_q = jnp.zeros((16,), jnp.float32)
        for f in range(F):
          e = e_v[rbase + f, :]
          acc_s = acc_s + e
          acc_q = acc_q + e * e
        d = acc_s * acc_s - acc_q
        wv1 = w_v[pl.ds(rbase, 16)]
        wv2 = w_v[pl.ds(rbase + 16, 16)]
        t = 0.5 * d + wv1 + jnp.where(wmask, wv2, 0.0)
        tbuf[r, pl.ds(0, 16)] = t
        return 0

      lax.fori_loop(0, 16, row, 0)
      acc = jnp.zeros((16,), jnp.float32)
      for dcol in range(16):
        col = plsc.load_gather(tbuf, [lane, jnp.full((16,), dcol, jnp.int32)])
        acc = acc + col
      out_v[pl.ds(c * CHUNK_ROWS + g * 16, 16)] = acc
      return 0

    lax.fori_loop(0, CHUNK_ROWS // 16, group, 0)

  fire(0, 0)
  for c in range(N_CHUNKS):
    slot = c % 2
    if c + 1 < N_CHUNKS:
      fire(1 - slot, c + 1)
    drain(slot)
    compute(slot, c)

  bias = bias_v[...]

  def sig(i, _):
    v = out_v[pl.ds(i * 16, 16)]
    z = v + bias
    out_v[pl.ds(i * 16, 16)] = 1.0 / (1.0 + jnp.exp(-z))
    return 0

  lax.fori_loop(0, ROWS_PER_W // 16, sig, 0)
  pltpu.sync_copy(out_v, out_hbm.at[pl.ds(wid * ROWS_PER_W, ROWS_PER_W)])


@functools.partial(jax.jit, static_argnames=())
def _fm_call(idx, emb_table, w_flat, bias16):
  mesh = plsc.VectorSubcoreMesh(core_axis_name="c", subcore_axis_name="s")
  run = pl.kernel(
      _fm_body,
      out_type=jax.ShapeDtypeStruct((B,), jnp.float32),
      mesh=mesh,
      compiler_params=pltpu.CompilerParams(
          needs_layout_passes=False, use_tc_tiling_on_sc=False),
      scratch_types=[
          pltpu.VMEM((CHUNK_IDX,), jnp.int32),
          pltpu.VMEM((CHUNK_IDX,), jnp.int32),
          pltpu.VMEM((CHUNK_IDX, D), jnp.float32),
          pltpu.VMEM((CHUNK_IDX, D), jnp.float32),
          pltpu.VMEM((CHUNK_IDX + W_PAD,), jnp.float32),
          pltpu.VMEM((CHUNK_IDX + W_PAD,), jnp.float32),
          pltpu.VMEM((ROWS_PER_W,), jnp.float32),
          pltpu.VMEM((16,), jnp.float32),
          pltpu.VMEM((16, 17), jnp.float32),
          pltpu.SemaphoreType.DMA,
          pltpu.SemaphoreType.DMA,
          pltpu.SemaphoreType.DMA,
          pltpu.SemaphoreType.DMA,
      ],
  )
  return run(idx, emb_table, w_flat, bias16)


def kernel(x, emb_table, w_table, bias):
  idx = x.reshape(-1)
  tail = emb_table[NBLK_FULL * 128:].reshape(-1)
  flat = _relayout_call(emb_table.T, tail)
  table = flat.reshape(NROW, D)
  w_flat = w_table.reshape(-1)
  bias16 = jnp.broadcast_to(bias, (16,))
  out = _fm_call(idx, table, w_flat, bias16)
  return out.reshape(B, 1)
